# baseline (speedup 1.0000x reference)
import functools

import functools

import jax
import jax.numpy as jnp
from jax import lax
from jax.experimental import pallas as pl
from jax.experimental.pallas import tpu as pltpu
from jax.experimental.pallas import tpu_sc as plsc

_COMMIT = 0.25


_R = 400
_C = 2048


def _dist_argmin_body(x_ref, w_ref, out_ref, wsq_ref):
    @pl.when(pl.program_id(0) == 0)
    def _():
        wb = w_ref[...]
        wsq_ref[...] = jnp.sum(wb * wb, axis=1).reshape(1, wb.shape[0])

    xb = x_ref[...]
    xsq = 0.25 * jnp.sum(xb * xb, axis=1, keepdims=True)
    s2 = lax.dot_general(xb, w_ref[...], (((1,), (1,)), ((), ())),
                         preferred_element_type=jnp.float32)
    d = (xsq + wsq_ref[...]) - s2
    di = lax.bitcast_convert_type(d, jnp.int32)
    rel = di - di[:, 0:1]
    lane = lax.broadcasted_iota(jnp.int32, d.shape, 1)
    key = jnp.left_shift(rel, 13) + lane
    kmin = jnp.min(key, axis=1, keepdims=True)
    out_ref[...] = jnp.bitwise_and(kmin, d.shape[1] - 1)


def _dist_argmin(x2, weight, blk_off, nblk):
    d_model = x2.shape[1]
    k = weight.shape[0]
    nh = nblk * _R
    out = pl.pallas_call(
        _dist_argmin_body,
        grid=(nblk,),
        in_specs=[
            pl.BlockSpec((_R, d_model), lambda i: (i + blk_off, 0)),
            pl.BlockSpec((k, d_model), lambda i: (0, 0)),
        ],
        out_specs=pl.BlockSpec((_R, 1), lambda i: (i, 0)),
        out_shape=jax.ShapeDtypeStruct((nh, 1), jnp.int32),
        scratch_shapes=[pltpu.VMEM((1, k), jnp.float32)],
    )(x2, weight)
    return out.reshape(nh)


_NW = 32


def _row_loop(rows_v, x_v, nrows, nq, init):
    def row_body(r, a):
        a = list(a)
        for q in range(nq):
            sl = pl.ds(q * 16, 16)
            zq = rows_v[r, sl]
            xx = x_v[r, sl]
            t = zq - xx
            rows_v[r, sl] = xx + t
            a[q % 4] = a[q % 4] + t * t
        return tuple(a)
    return lax.fori_loop(0, nrows, row_body, init)


def _make_sc_stage2(lo, nh, bpw, sub, tails, d_model):
    nsub = bpw // sub
    nq = d_model // 16
    tlen = max((ln for _, _, ln in tails), default=16)

    def body(w_hbm, idx_hbm, x_hbm, out_hbm, part_hbm,
             idx0, idx1, rows0, rows1, x0, x1,
             tidx, trows, tx, acc_v,
             g0, g1, xs0, xs1, os0, os1, tsem):
        wid = lax.axis_index("s") * 2 + lax.axis_index("c")
        base = wid * bpw
        zero4 = tuple(jnp.zeros((16,), jnp.float32) for _ in range(4))
        acc_v[...] = jnp.zeros((16,), jnp.float32)

        for twid, tst, tln in tails:
            @pl.when(wid == twid)
            def _(tst=tst, tln=tln):
                pltpu.sync_copy(idx_hbm.at[pl.ds(tst, tln)],
                                tidx.at[pl.ds(0, tln)])
                pltpu.async_copy(w_hbm.at[tidx.at[pl.ds(0, tln)]],
                                 trows.at[pl.ds(0, tln)], tsem).wait()
                pltpu.sync_copy(x_hbm.at[pl.ds(lo + tst, tln)],
                                tx.at[pl.ds(0, tln)])
                ta = _row_loop(trows, tx, tln, nq, zero4)
                acc_v[...] = acc_v[...] + ((ta[0] + ta[1]) + (ta[2] + ta[3]))
                pltpu.sync_copy(trows.at[pl.ds(0, tln)],
                                out_hbm.at[pl.ds(tst, tln)])

        bufs = [(idx0, rows0, x0, g0, xs0, os0),
                (idx1, rows1, x1, g1, xs1, os1)]

        def start_fetch(cc):
            idx_v, rows_v, x_v, gsem, xsem, _ = bufs[cc % 2]
            st = base + cc * sub
            pltpu.sync_copy(idx_hbm.at[pl.ds(st, sub)], idx_v)
            gh = pltpu.async_copy(w_hbm.at[idx_v], rows_v, gsem)
            xh = pltpu.async_copy(x_hbm.at[pl.ds(lo + st, sub)], x_v, xsem)
            return gh, xh

        fetch_h = {0: start_fetch(0)}
        out_h = {}
        accs = zero4
        for cc in range(nsub):
            idx_v, rows_v, x_v, _, _, osem = bufs[cc % 2]
            if cc + 1 < nsub:
                if cc >= 1:
                    out_h.pop(cc - 1).wait()
                fetch_h[cc + 1] = start_fetch(cc + 1)
            gh, xh = fetch_h.pop(cc)
            gh.wait()
            xh.wait()
            accs = _row_loop(rows_v, x_v, sub, nq, accs)
            out_h[cc] = pltpu.async_copy(
                rows_v, out_hbm.at[pl.ds(base + cc * sub, sub)], osem)
        for cc in sorted(out_h):
            out_h[cc].wait()
        acc_v[...] = acc_v[...] + ((accs[0] + accs[1]) + (accs[2] + accs[3]))
        pltpu.sync_copy(acc_v, part_hbm.at[wid])

    mesh = plsc.VectorSubcoreMesh(core_axis_name="c", subcore_axis_name="s")
    return pl.kernel(
        body, mesh=mesh,
        out_type=[jax.ShapeDtypeStruct((nh, d_model), jnp.float32),
                  jax.ShapeDtypeStruct((_NW, 16), jnp.float32)],
        scratch_types=[
            pltpu.VMEM((sub,), jnp.int32),
            pltpu.VMEM((sub,), jnp.int32),
            pltpu.VMEM((sub, d_model), jnp.float32),
            pltpu.VMEM((sub, d_model), jnp.float32),
            pltpu.VMEM((sub, d_model), jnp.float32),
            pltpu.VMEM((sub, d_model), jnp.float32),
            pltpu.VMEM((tlen,), jnp.int32),
            pltpu.VMEM((tlen, d_model), jnp.float32),
            pltpu.VMEM((tlen, d_model), jnp.float32),
            pltpu.VMEM((16,), jnp.float32),
            pltpu.SemaphoreType.DMA,
            pltpu.SemaphoreType.DMA,
            pltpu.SemaphoreType.DMA,
            pltpu.SemaphoreType.DMA,
            pltpu.SemaphoreType.DMA,
            pltpu.SemaphoreType.DMA,
            pltpu.SemaphoreType.DMA,
        ],
    )


def kernel(x, edge_index, weight):
    n, d_model = x.shape
    x2 = 2.0 * x
    idx_a = _dist_argmin(x2, weight, 0, 13)
    sc_a = _make_sc_stage2(0, 5200, 160, 80, [(0, 5120, 80)], d_model)
    za, pa = sc_a(weight, idx_a, x)
    idx_b = _dist_argmin(x2, weight, 13, 12)
    sc_b = _make_sc_stage2(5200, 4800, 144, 72,
                           [(0, 4608, 96), (1, 4704, 96)], d_model)
    zb, pb = sc_b(weight, idx_b, x)
    z_q_st = jnp.concatenate([za, zb])
    idx = jnp.concatenate([idx_a, idx_b])
    m = (jnp.sum(pa) + jnp.sum(pb)) / (n * d_model)
    loss = m + _COMMIT * m
    return (z_q_st, edge_index, loss, idx)


# speedup vs baseline: 1.0903x; 1.0903x over previous
import jax
import jax.numpy as jnp
from jax import lax
from jax.experimental import pallas as pl
from jax.experimental.pallas import tpu as pltpu
from jax.experimental.pallas import tpu_sc as plsc

_COMMIT = 0.25


_R = 400


def _dist_argmin_body(x_ref, w_ref, out_ref, wsq_ref):
    @pl.when(pl.program_id(0) == 0)
    def _():
        wb = w_ref[...]
        wsq_ref[...] = jnp.sum(wb * wb, axis=1).reshape(1, wb.shape[0])

    xb = x_ref[...]
    xsq = 0.25 * jnp.sum(xb * xb, axis=1, keepdims=True)
    s2 = lax.dot_general(xb, w_ref[...], (((1,), (1,)), ((), ())),
                         preferred_element_type=jnp.float32)
    d = (xsq + wsq_ref[...]) - s2
    di = lax.bitcast_convert_type(d, jnp.int32)
    rel = di - di[:, 0:1]
    lane = lax.broadcasted_iota(jnp.int32, d.shape, 1)
    key = jnp.left_shift(rel, 13) + lane
    kmin = jnp.min(key, axis=1, keepdims=True)
    out_ref[...] = jnp.bitwise_and(kmin, d.shape[1] - 1)


def _dist_argmin(x, weight):
    n, d_model = x.shape
    k = weight.shape[0]
    nblk = n // _R
    out = pl.pallas_call(
        _dist_argmin_body,
        grid=(nblk,),
        in_specs=[
            pl.BlockSpec((_R, d_model), lambda i: (i, 0)),
            pl.BlockSpec((k, d_model), lambda i: (0, 0)),
        ],
        out_specs=pl.BlockSpec((_R, 1), lambda i: (i, 0)),
        out_shape=jax.ShapeDtypeStruct((n, 1), jnp.int32),
        scratch_shapes=[pltpu.VMEM((1, k), jnp.float32)],
    )(2.0 * x, weight)
    return out.reshape(n)


_NW = 32
_BPW = 312
_SUB = 104
_NSUB = _BPW // _SUB
_TAIL = 16


def _row_loop(rows_v, x_v, nrows, nq, init):
    def row_body(r, a):
        a = list(a)
        for q in range(nq):
            sl = pl.ds(q * 16, 16)
            zq = rows_v[r, sl]
            xx = x_v[r, sl]
            t = zq - xx
            rows_v[r, sl] = xx + t
            a[q % 4] = a[q % 4] + t * t
        return tuple(a)
    return lax.fori_loop(0, nrows, row_body, init)


def _sc_stage2_body(w_hbm, idx_hbm, x_hbm, out_hbm, part_hbm,
                    idx0, idx1, rows0, rows1, x0, x1,
                    tidx, trows, tx, acc_v,
                    g0, g1, xs0, xs1, os0, os1, tsem):
    d_model = w_hbm.shape[1]
    nq = d_model // 16
    wid = lax.axis_index("s") * 2 + lax.axis_index("c")
    base = wid * _BPW
    zero4 = tuple(jnp.zeros((16,), jnp.float32) for _ in range(4))
    acc_v[...] = jnp.zeros((16,), jnp.float32)

    @pl.when(wid == 0)
    def _():
        st = _NW * _BPW
        pltpu.sync_copy(idx_hbm.at[pl.ds(st, _TAIL)], tidx)
        pltpu.async_copy(w_hbm.at[tidx], trows, tsem).wait()
        pltpu.sync_copy(x_hbm.at[pl.ds(st, _TAIL)], tx)
        ta = _row_loop(trows, tx, _TAIL, nq, zero4)
        acc_v[...] = (ta[0] + ta[1]) + (ta[2] + ta[3])
        pltpu.sync_copy(trows, out_hbm.at[pl.ds(st, _TAIL)])

    bufs = [(idx0, rows0, x0, g0, xs0, os0),
            (idx1, rows1, x1, g1, xs1, os1)]

    def start_fetch(cc):
        idx_v, rows_v, x_v, gsem, xsem, _ = bufs[cc % 2]
        st = base + cc * _SUB
        pltpu.sync_copy(idx_hbm.at[pl.ds(st, _SUB)], idx_v)
        gh = pltpu.async_copy(w_hbm.at[idx_v], rows_v, gsem)
        xh = pltpu.async_copy(x_hbm.at[pl.ds(st, _SUB)], x_v, xsem)
        return gh, xh

    fetch_h = {0: start_fetch(0)}
    out_h = {}
    accs = zero4
    for cc in range(_NSUB):
        idx_v, rows_v, x_v, _, _, osem = bufs[cc % 2]
        if cc + 1 < _NSUB:
            if cc >= 1:
                out_h.pop(cc - 1).wait()
            fetch_h[cc + 1] = start_fetch(cc + 1)
        gh, xh = fetch_h.pop(cc)
        gh.wait()
        xh.wait()
        accs = _row_loop(rows_v, x_v, _SUB, nq, accs)
        out_h[cc] = pltpu.async_copy(
            rows_v, out_hbm.at[pl.ds(base + cc * _SUB, _SUB)], osem)
    for cc in sorted(out_h):
        out_h[cc].wait()
    acc_v[...] = acc_v[...] + ((accs[0] + accs[1]) + (accs[2] + accs[3]))
    pltpu.sync_copy(acc_v, part_hbm.at[wid])


def _sc_stage2(weight, idx, x):
    d_model = weight.shape[1]
    n = idx.shape[0]
    mesh = plsc.VectorSubcoreMesh(core_axis_name="c", subcore_axis_name="s")
    fn = pl.kernel(
        _sc_stage2_body, mesh=mesh,
        out_type=[jax.ShapeDtypeStruct((n, d_model), jnp.float32),
                  jax.ShapeDtypeStruct((_NW, 16), jnp.float32)],
        scratch_types=[
            pltpu.VMEM((_SUB,), jnp.int32),
            pltpu.VMEM((_SUB,), jnp.int32),
            pltpu.VMEM((_SUB, d_model), jnp.float32),
            pltpu.VMEM((_SUB, d_model), jnp.float32),
            pltpu.VMEM((_SUB, d_model), jnp.float32),
            pltpu.VMEM((_SUB, d_model), jnp.float32),
            pltpu.VMEM((_TAIL,), jnp.int32),
            pltpu.VMEM((_TAIL, d_model), jnp.float32),
            pltpu.VMEM((_TAIL, d_model), jnp.float32),
            pltpu.VMEM((16,), jnp.float32),
            pltpu.SemaphoreType.DMA,
            pltpu.SemaphoreType.DMA,
            pltpu.SemaphoreType.DMA,
            pltpu.SemaphoreType.DMA,
            pltpu.SemaphoreType.DMA,
            pltpu.SemaphoreType.DMA,
            pltpu.SemaphoreType.DMA,
        ],
    )
    return fn(weight, idx, x)


def kernel(x, edge_index, weight):
    n, d_model = x.shape
    idx = _dist_argmin(x, weight)

    z_q_st, partials = _sc_stage2(weight, idx, x)
    m = jnp.sum(partials) / (n * d_model)
    loss = m + _COMMIT * m
    return (z_q_st, edge_index, loss, idx)
